# Initial kernel scaffold; baseline (speedup 1.0000x reference)
import jax
import jax.numpy as jnp
from jax.experimental import pallas as pl


def kernel(edge_distance, source_element, target_element, W1, b1, src_emb, tgt_emb, W2, b2):
    raise NotImplementedError("write your pallas kernel here")


# speedup vs baseline: 2.1052x; 2.1052x over previous
import functools

import jax
import jax.numpy as jnp
import numpy as np
from jax import lax
from jax.experimental import pallas as pl

E = 320000
C = 128
NG = 50
MAX_ELEM = 100
GS_START, GS_STOP = 0.0, 8.0

_OFFSETS = np.linspace(GS_START, GS_STOP, NG).astype(np.float32)
_COEFF = np.float32(-0.5 / (_OFFSETS[1] - _OFFSETS[0]) ** 2)

BLOCK = 2560


def _edge_block_kernel(dist_ref, src_ref, tgt_ref, w1t_ref, b1_ref,
                       semb_ref, temb_ref, w2t_ref, b2_ref, out_ref):
    b = dist_ref.shape[0]
    dist = dist_ref[...]
    step = np.float32((GS_STOP - GS_START) / (NG - 1))
    offs = lax.broadcasted_iota(jnp.int32, (1, NG), 1).astype(jnp.float32) * step + GS_START
    d = dist - offs
    gauss = jnp.exp(_COEFF * d * d)

    iota = lax.broadcasted_iota(jnp.int32, (b, MAX_ELEM), 1)
    oh_s = (src_ref[...] == iota).astype(jnp.float32)
    oh_t = (tgt_ref[...] == iota).astype(jnp.float32)

    h = jnp.dot(gauss, w1t_ref[...], preferred_element_type=jnp.float32)
    h += jnp.dot(oh_s, semb_ref[...], preferred_element_type=jnp.float32)
    h += jnp.dot(oh_t, temb_ref[...], preferred_element_type=jnp.float32)
    h += b1_ref[...]
    h = h * jax.nn.sigmoid(h)
    o = jnp.dot(h, w2t_ref[...], preferred_element_type=jnp.float32)
    o += b2_ref[...]
    out_ref[...] = o * jax.nn.sigmoid(o)


@functools.partial(jax.jit, static_argnames=())
def kernel(edge_distance, source_element, target_element, W1, b1,
           src_emb, tgt_emb, W2, b2):
    dist = edge_distance.reshape(E, 1)
    src = source_element.astype(jnp.int32).reshape(E, 1)
    tgt = target_element.astype(jnp.int32).reshape(E, 1)
    w1t = W1.T
    w2t = W2.T
    b1r = b1.reshape(1, C)
    b2r = b2.reshape(1, C)

    nb = E // BLOCK
    edge_spec = pl.BlockSpec((BLOCK, 1), lambda i: (i, 0))
    full = lambda shape: pl.BlockSpec(shape, lambda i: (0, 0))

    out = pl.pallas_call(
        _edge_block_kernel,
        grid=(nb,),
        in_specs=[
            edge_spec,
            edge_spec,
            edge_spec,
            full((NG, C)),
            full((1, C)),
            full((MAX_ELEM, C)),
            full((MAX_ELEM, C)),
            full((C, C)),
            full((1, C)),
        ],
        out_specs=pl.BlockSpec((BLOCK, C), lambda i: (i, 0)),
        out_shape=jax.ShapeDtypeStruct((E, C), jnp.float32),
    )(dist, src, tgt, w1t, b1r, src_emb, tgt_emb, w2t, b2r)
    return out


# speedup vs baseline: 2.2771x; 1.0817x over previous
import functools

import jax
import jax.numpy as jnp
import numpy as np
from jax import lax
from jax.experimental import pallas as pl

E = 320000
C = 128
NG = 50
MAX_ELEM = 100
GS_START, GS_STOP = 0.0, 8.0

_OFFSETS = np.linspace(GS_START, GS_STOP, NG).astype(np.float32)
_COEFF = np.float32(-0.5 / (_OFFSETS[1] - _OFFSETS[0]) ** 2)

BLOCK = 6400


def _edge_block_kernel(dist_ref, src_ref, tgt_ref, w1t_ref, b1_ref,
                       semb_ref, temb_ref, w2t_ref, b2_ref, out_ref):
    b = dist_ref.shape[0]
    dist = dist_ref[...]
    step = np.float32((GS_STOP - GS_START) / (NG - 1))
    offs = lax.broadcasted_iota(jnp.int32, (1, NG), 1).astype(jnp.float32) * step + GS_START
    d = dist - offs
    gauss = jnp.exp(_COEFF * d * d)

    iota = lax.broadcasted_iota(jnp.int32, (b, MAX_ELEM), 1)
    oh_s = (src_ref[...] == iota).astype(jnp.float32)
    oh_t = (tgt_ref[...] == iota).astype(jnp.float32)

    h = jnp.dot(gauss, w1t_ref[...], preferred_element_type=jnp.float32)
    h += jnp.dot(oh_s, semb_ref[...], preferred_element_type=jnp.float32)
    h += jnp.dot(oh_t, temb_ref[...], preferred_element_type=jnp.float32)
    h += b1_ref[...]
    h = h * jax.nn.sigmoid(h)
    o = jnp.dot(h, w2t_ref[...], preferred_element_type=jnp.float32)
    o += b2_ref[...]
    out_ref[...] = o * jax.nn.sigmoid(o)


@functools.partial(jax.jit, static_argnames=())
def kernel(edge_distance, source_element, target_element, W1, b1,
           src_emb, tgt_emb, W2, b2):
    dist = edge_distance.reshape(E, 1)
    src = source_element.astype(jnp.int32).reshape(E, 1)
    tgt = target_element.astype(jnp.int32).reshape(E, 1)
    w1t = W1.T
    w2t = W2.T
    b1r = b1.reshape(1, C)
    b2r = b2.reshape(1, C)

    nb = E // BLOCK
    edge_spec = pl.BlockSpec((BLOCK, 1), lambda i: (i, 0))
    full = lambda shape: pl.BlockSpec(shape, lambda i: (0, 0))

    out = pl.pallas_call(
        _edge_block_kernel,
        grid=(nb,),
        in_specs=[
            edge_spec,
            edge_spec,
            edge_spec,
            full((NG, C)),
            full((1, C)),
            full((MAX_ELEM, C)),
            full((MAX_ELEM, C)),
            full((C, C)),
            full((1, C)),
        ],
        out_specs=pl.BlockSpec((BLOCK, C), lambda i: (i, 0)),
        out_shape=jax.ShapeDtypeStruct((E, C), jnp.float32),
    )(dist, src, tgt, w1t, b1r, src_emb, tgt_emb, w2t, b2r)
    return out


# speedup vs baseline: 2.3084x; 1.0137x over previous
import functools

import jax
import jax.numpy as jnp
import numpy as np
from jax import lax
from jax.experimental import pallas as pl

E = 320000
C = 128
NG = 50
MAX_ELEM = 100
GS_START, GS_STOP = 0.0, 8.0

_OFFSETS = np.linspace(GS_START, GS_STOP, NG).astype(np.float32)
_COEFF = np.float32(-0.5 / (_OFFSETS[1] - _OFFSETS[0]) ** 2)

BLOCK = 12800


def _edge_block_kernel(dist_ref, src_ref, tgt_ref, w1t_ref, b1_ref,
                       semb_ref, temb_ref, w2t_ref, b2_ref, out_ref):
    b = dist_ref.shape[0]
    dist = dist_ref[...]
    step = np.float32((GS_STOP - GS_START) / (NG - 1))
    offs = lax.broadcasted_iota(jnp.int32, (1, NG), 1).astype(jnp.float32) * step + GS_START
    d = dist - offs
    gauss = jnp.exp(_COEFF * d * d)

    iota = lax.broadcasted_iota(jnp.int32, (b, MAX_ELEM), 1)
    oh_s = (src_ref[...] == iota).astype(jnp.float32)
    oh_t = (tgt_ref[...] == iota).astype(jnp.float32)

    h = jnp.dot(gauss, w1t_ref[...], preferred_element_type=jnp.float32)
    h += jnp.dot(oh_s, semb_ref[...], preferred_element_type=jnp.float32)
    h += jnp.dot(oh_t, temb_ref[...], preferred_element_type=jnp.float32)
    h += b1_ref[...]
    h = h * jax.nn.sigmoid(h)
    o = jnp.dot(h, w2t_ref[...], preferred_element_type=jnp.float32)
    o += b2_ref[...]
    out_ref[...] = o * jax.nn.sigmoid(o)


@functools.partial(jax.jit, static_argnames=())
def kernel(edge_distance, source_element, target_element, W1, b1,
           src_emb, tgt_emb, W2, b2):
    dist = edge_distance.reshape(E, 1)
    src = source_element.astype(jnp.int32).reshape(E, 1)
    tgt = target_element.astype(jnp.int32).reshape(E, 1)
    w1t = W1.T
    w2t = W2.T
    b1r = b1.reshape(1, C)
    b2r = b2.reshape(1, C)

    nb = E // BLOCK
    edge_spec = pl.BlockSpec((BLOCK, 1), lambda i: (i, 0))
    full = lambda shape: pl.BlockSpec(shape, lambda i: (0, 0))

    out = pl.pallas_call(
        _edge_block_kernel,
        grid=(nb,),
        in_specs=[
            edge_spec,
            edge_spec,
            edge_spec,
            full((NG, C)),
            full((1, C)),
            full((MAX_ELEM, C)),
            full((MAX_ELEM, C)),
            full((C, C)),
            full((1, C)),
        ],
        out_specs=pl.BlockSpec((BLOCK, C), lambda i: (i, 0)),
        out_shape=jax.ShapeDtypeStruct((E, C), jnp.float32),
    )(dist, src, tgt, w1t, b1r, src_emb, tgt_emb, w2t, b2r)
    return out
